# baseline (speedup 1.0000x reference)
import functools

import jax
import jax.numpy as jnp
from jax import lax
from jax.experimental import pallas as pl
from jax.experimental.pallas import tpu as pltpu

NBUF = 4
BM = 200


def _gcn_body(adj_hbm, x_ref, w_ref, b_ref, out_ref, buf, xw_ref, sem):
    n_rows = x_ref.shape[0]
    nchunks = adj_hbm.shape[0] // BM

    def copy_for(j, slot):
        return pltpu.make_async_copy(
            adj_hbm.at[pl.ds(j * BM, BM), :],
            buf.at[slot],
            sem.at[slot],
        )

    for j in range(NBUF):
        copy_for(j, j).start()

    xw_ref[...] = jnp.dot(x_ref[...], w_ref[...],
                          preferred_element_type=jnp.float32)

    def loop_body(j, carry):
        slot = lax.rem(j, NBUF)
        copy_for(j, slot).wait()
        out_ref[pl.ds(j * BM, BM), :] = jnp.dot(
            buf[slot], xw_ref[...],
            preferred_element_type=jnp.float32) + b_ref[...]

        @pl.when(j + NBUF < nchunks)
        def _():
            copy_for(j + NBUF, slot).start()

        return carry

    lax.fori_loop(0, nchunks, loop_body, 0, unroll=False)


def kernel(input, adj, weight, bias):
    n_rows, f_in = input.shape
    f_out = weight.shape[1]
    n_dst = adj.shape[0]

    out = pl.pallas_call(
        _gcn_body,
        in_specs=[
            pl.BlockSpec(memory_space=pltpu.MemorySpace.HBM),
            pl.BlockSpec(memory_space=pltpu.MemorySpace.VMEM),
            pl.BlockSpec(memory_space=pltpu.MemorySpace.VMEM),
            pl.BlockSpec(memory_space=pltpu.MemorySpace.VMEM),
        ],
        out_specs=pl.BlockSpec(memory_space=pltpu.MemorySpace.VMEM),
        out_shape=jax.ShapeDtypeStruct((n_dst, f_out), jnp.float32),
        scratch_shapes=[
            pltpu.VMEM((NBUF, BM, n_rows), jnp.float32),
            pltpu.VMEM((n_rows, f_out), jnp.float32),
            pltpu.SemaphoreType.DMA((NBUF,)),
        ],
        compiler_params=pltpu.CompilerParams(
            vmem_limit_bytes=64 * 1024 * 1024,
        ),
    )(adj, input, weight, bias.reshape(1, f_out))
    return out
